# Initial kernel scaffold; baseline (speedup 1.0000x reference)
import jax
import jax.numpy as jnp
from jax.experimental import pallas as pl


def kernel(x, edge_index, W1, b1, W2, b2):
    raise NotImplementedError("write your pallas kernel here")


# speedup vs baseline: 19.2403x; 19.2403x over previous
import functools

import jax
import jax.numpy as jnp
from jax import lax
from jax.experimental import pallas as pl
from jax.experimental.pallas import tpu as pltpu
from jax.experimental.pallas import tpu_sc as plsc

_NC = 2
_NS = 16
_NW = _NC * _NS
_W = 80
_RB = 1000


def _mesh():
    return plsc.VectorSubcoreMesh(core_axis_name="c", subcore_axis_name="s")


_DW = 16


def _sc_degree(dst3, zeros_col, ones_col, compiler_params=None):
    n, dw = zeros_col.shape
    nwin = dst3.shape[1]

    @functools.partial(
        pl.kernel,
        out_type=jax.ShapeDtypeStruct((_NC, n, dw), jnp.float32),
        mesh=_mesh(),
        compiler_params=compiler_params,
        scratch_types=[
            pltpu.VMEM_SHARED((n, dw), jnp.float32),
            pltpu.VMEM((nwin, _W), jnp.int32),
            pltpu.VMEM((_W, dw), jnp.float32),
        ],
    )
    def deg_kernel(dst_hbm, zeros_hbm, ones_hbm, out_hbm, acc_sh, dst_v, ones_v):
        cid = lax.axis_index("c")
        sid = lax.axis_index("s")
        wid = cid * _NS + sid

        pltpu.sync_copy(ones_hbm, ones_v)

        @pl.when(sid < n // _RB)
        def _():
            sl = pl.ds(sid * _RB, _RB)
            pltpu.sync_copy(zeros_hbm.at[sl], acc_sh.at[sl])

        plsc.subcore_barrier()

        pltpu.sync_copy(dst_hbm.at[wid], dst_v)

        @pl.loop(0, nwin)
        def _(w):
            pltpu.sync_copy(ones_v, acc_sh.at[dst_v.at[w]], add=True)

        plsc.subcore_barrier()

        @pl.when(sid < n // _RB)
        def _():
            sl = pl.ds(sid * _RB, _RB)
            pltpu.sync_copy(acc_sh.at[sl], out_hbm.at[cid, sl])

    return deg_kernel(dst3, zeros_col, ones_col)


def _sc_aggregate(p, src3, dst3):
    n, d = p.shape
    nwin = src3.shape[1]

    @functools.partial(
        pl.kernel,
        out_type=jax.ShapeDtypeStruct((_NC, n, d), jnp.float32),
        mesh=_mesh(),
        scratch_types=[
            pltpu.VMEM_SHARED((n, d), jnp.float32),
            pltpu.VMEM((nwin, _W), jnp.int32),
            pltpu.VMEM((nwin, _W), jnp.int32),
            pltpu.VMEM((_W, d), jnp.float32),
            pltpu.SemaphoreType.DMA,
        ],
    )
    def agg_kernel(p_hbm, src_hbm, dst_hbm, out_hbm, acc_sh, src_v, dst_v, rows_v, sem):
        cid = lax.axis_index("c")
        sid = lax.axis_index("s")
        wid = cid * _NS + sid

        @pl.when(sid < n // _RB)
        def _():
            sl = pl.ds(sid * _RB, _RB)
            pltpu.sync_copy(p_hbm.at[sl], acc_sh.at[sl])

        pltpu.sync_copy(src_hbm.at[wid], src_v)
        pltpu.sync_copy(dst_hbm.at[wid], dst_v)
        plsc.subcore_barrier()

        @pl.loop(0, nwin)
        def _(w):
            pltpu.async_copy(p_hbm.at[src_v.at[w]], rows_v, sem).wait()
            pltpu.sync_copy(rows_v, acc_sh.at[dst_v.at[w]], add=True)

        plsc.subcore_barrier()

        @pl.when(sid < n // _RB)
        def _():
            sl = pl.ds(sid * _RB, _RB)
            pltpu.sync_copy(acc_sh.at[sl], out_hbm.at[cid, sl])

    return agg_kernel(p, src3, dst3)


def _dinv_block(deg_ref, i):
    dparts = deg_ref[:, pl.ds(i * _RB, _RB), 0:1]
    deg = dparts[0] + dparts[1] + 1.0
    return lax.rsqrt(deg)


def _tc_matmul(x, w):
    n, din = x.shape
    dh = w.shape[1]

    def body(x_ref, w_ref, o_ref):
        o_ref[...] = lax.dot_general(
            x_ref[...], w_ref[...], (((1,), (0,)), ((), ())),
            precision=lax.Precision.HIGHEST, preferred_element_type=jnp.float32)

    return pl.pallas_call(
        body,
        grid=(n // _RB,),
        in_specs=[
            pl.BlockSpec((_RB, din), lambda i: (i, 0)),
            pl.BlockSpec((din, dh), lambda i: (0, 0)),
        ],
        out_specs=pl.BlockSpec((_RB, dh), lambda i: (i, 0)),
        out_shape=jax.ShapeDtypeStruct((n, dh), jnp.float32),
    )(x, w)


def _tc_scale(h, deg_parts):
    n, d = h.shape

    def body(h_ref, deg_ref, o_ref):
        i = pl.program_id(0)
        o_ref[...] = h_ref[...] * _dinv_block(deg_ref, i)

    return pl.pallas_call(
        body,
        grid=(n // _RB,),
        in_specs=[
            pl.BlockSpec((_RB, d), lambda i: (i, 0)),
            pl.BlockSpec((_NC, n, _DW), lambda i: (0, 0, 0)),
        ],
        out_specs=pl.BlockSpec((_RB, d), lambda i: (i, 0)),
        out_shape=jax.ShapeDtypeStruct((n, d), jnp.float32),
    )(h, deg_parts)


def _tc_mid(s_parts, p1, deg_parts, b1, w2):
    n, d = p1.shape
    dh = w2.shape[1]

    def body(s_ref, p_ref, deg_ref, b_ref, w_ref, o_ref):
        i = pl.program_id(0)
        dinv = _dinv_block(deg_ref, i)
        sv = s_ref[...]
        s = sv[0] + sv[1] - p_ref[...]
        t = jnp.maximum(s * dinv + b_ref[...], 0.0)
        h2 = lax.dot_general(
            t, w_ref[...], (((1,), (0,)), ((), ())),
            precision=lax.Precision.HIGHEST, preferred_element_type=jnp.float32)
        o_ref[...] = h2 * dinv

    return pl.pallas_call(
        body,
        grid=(n // _RB,),
        in_specs=[
            pl.BlockSpec((_NC, _RB, d), lambda i: (0, i, 0)),
            pl.BlockSpec((_RB, d), lambda i: (i, 0)),
            pl.BlockSpec((_NC, n, _DW), lambda i: (0, 0, 0)),
            pl.BlockSpec((1, d), lambda i: (0, 0)),
            pl.BlockSpec((d, dh), lambda i: (0, 0)),
        ],
        out_specs=pl.BlockSpec((_RB, dh), lambda i: (i, 0)),
        out_shape=jax.ShapeDtypeStruct((n, dh), jnp.float32),
    )(s_parts, p1, deg_parts, b1, w2)


def _tc_final(s_parts, p2, deg_parts, b2):
    n, d = p2.shape

    def body(s_ref, p_ref, deg_ref, b_ref, o_ref):
        i = pl.program_id(0)
        dinv = _dinv_block(deg_ref, i)
        sv = s_ref[...]
        s = sv[0] + sv[1] - p_ref[...]
        o_ref[...] = s * dinv + b_ref[...]

    return pl.pallas_call(
        body,
        grid=(n // _RB,),
        in_specs=[
            pl.BlockSpec((_NC, _RB, d), lambda i: (0, i, 0)),
            pl.BlockSpec((_RB, d), lambda i: (i, 0)),
            pl.BlockSpec((_NC, n, _DW), lambda i: (0, 0, 0)),
            pl.BlockSpec((1, d), lambda i: (0, 0)),
        ],
        out_specs=pl.BlockSpec((_RB, d), lambda i: (i, 0)),
        out_shape=jax.ShapeDtypeStruct((n, d), jnp.float32),
    )(s_parts, p2, deg_parts, b2)


def kernel(x, edge_index, W1, b1, W2, b2):
    n = x.shape[0]
    e = edge_index.shape[1]
    nwin = e // (_NW * _W)

    src3 = edge_index[0].reshape(_NW, nwin, _W)
    dst3 = edge_index[1].reshape(_NW, nwin, _W)
    zeros_col = jnp.zeros((n, _DW), jnp.float32)
    ones_col = jnp.ones((_W, _DW), jnp.float32)

    deg_parts = _sc_degree(
        dst3, zeros_col, ones_col,
        compiler_params=pltpu.CompilerParams(use_tc_tiling_on_sc=False),
    )
    h1 = _tc_matmul(x, W1)
    p1 = _tc_scale(h1, deg_parts)
    s1 = _sc_aggregate(p1, src3, dst3)
    p2 = _tc_mid(s1, p1, deg_parts, b1.reshape(1, -1), W2)
    s2 = _sc_aggregate(p2, src3, dst3)
    return _tc_final(s2, p2, deg_parts, b2.reshape(1, -1))


# speedup vs baseline: 28.7786x; 1.4957x over previous
import functools

import jax
import jax.numpy as jnp
from jax import lax
from jax.experimental import pallas as pl
from jax.experimental.pallas import tpu as pltpu
from jax.experimental.pallas import tpu_sc as plsc

_NC = 2
_NS = 16
_NW = _NC * _NS
_W = 80
_RB = 1000


def _mesh():
    return plsc.VectorSubcoreMesh(core_axis_name="c", subcore_axis_name="s")


_DW = 16


def _sc_degree(dst3, zeros_col, ones_col, compiler_params=None):
    n, dw = zeros_col.shape
    nwin = dst3.shape[1]

    @functools.partial(
        pl.kernel,
        out_type=jax.ShapeDtypeStruct((_NC, n, dw), jnp.float32),
        mesh=_mesh(),
        compiler_params=compiler_params,
        scratch_types=[
            pltpu.VMEM_SHARED((n, dw), jnp.float32),
            pltpu.VMEM((nwin, _W), jnp.int32),
            pltpu.VMEM((_W, dw), jnp.float32),
        ],
    )
    def deg_kernel(dst_hbm, zeros_hbm, ones_hbm, out_hbm, acc_sh, dst_v, ones_v):
        cid = lax.axis_index("c")
        sid = lax.axis_index("s")
        wid = cid * _NS + sid

        pltpu.sync_copy(ones_hbm, ones_v)

        @pl.when(sid < n // _RB)
        def _():
            sl = pl.ds(sid * _RB, _RB)
            pltpu.sync_copy(zeros_hbm.at[sl], acc_sh.at[sl])

        plsc.subcore_barrier()

        pltpu.sync_copy(dst_hbm.at[wid], dst_v)

        @pl.loop(0, nwin)
        def _(w):
            pltpu.sync_copy(ones_v, acc_sh.at[dst_v.at[w]], add=True)

        plsc.subcore_barrier()

        @pl.when(sid < n // _RB)
        def _():
            sl = pl.ds(sid * _RB, _RB)
            pltpu.sync_copy(acc_sh.at[sl], out_hbm.at[cid, sl])

    return deg_kernel(dst3, zeros_col, ones_col)


def _sc_aggregate(p, src3, dst3):
    n, d = p.shape
    nwin = src3.shape[1]

    @functools.partial(
        pl.kernel,
        out_type=jax.ShapeDtypeStruct((_NC, n, d), jnp.float32),
        mesh=_mesh(),
        compiler_params=pltpu.CompilerParams(use_tc_tiling_on_sc=False),
        scratch_types=[
            pltpu.VMEM_SHARED((n, d), jnp.float32),
            pltpu.VMEM((nwin, _W), jnp.int32),
            pltpu.VMEM((nwin, _W), jnp.int32),
            pltpu.VMEM((_W, d), jnp.float32),
            pltpu.VMEM((_W, d), jnp.float32),
            pltpu.SemaphoreType.DMA,
            pltpu.SemaphoreType.DMA,
        ],
    )
    def agg_kernel(p_hbm, src_hbm, dst_hbm, out_hbm, acc_sh, src_v, dst_v,
                   rows0, rows1, sem0, sem1):
        cid = lax.axis_index("c")
        sid = lax.axis_index("s")
        wid = cid * _NS + sid

        @pl.when(sid < n // _RB)
        def _():
            sl = pl.ds(sid * _RB, _RB)
            pltpu.sync_copy(p_hbm.at[sl], acc_sh.at[sl])

        pltpu.sync_copy(src_hbm.at[wid], src_v)
        pltpu.sync_copy(dst_hbm.at[wid], dst_v)
        plsc.subcore_barrier()

        pltpu.async_copy(p_hbm.at[src_v.at[0]], rows0, sem0)

        @pl.loop(0, nwin - 1, step=2)
        def _(w):
            pltpu.async_copy(p_hbm.at[src_v.at[w + 1]], rows1, sem1)
            pltpu.make_async_copy(p_hbm.at[src_v.at[w]], rows0, sem0).wait()
            pltpu.sync_copy(rows0, acc_sh.at[dst_v.at[w]], add=True)
            pltpu.async_copy(p_hbm.at[src_v.at[w + 2]], rows0, sem0)
            pltpu.make_async_copy(p_hbm.at[src_v.at[w + 1]], rows1, sem1).wait()
            pltpu.sync_copy(rows1, acc_sh.at[dst_v.at[w + 1]], add=True)

        pltpu.make_async_copy(p_hbm.at[src_v.at[nwin - 1]], rows0, sem0).wait()
        pltpu.sync_copy(rows0, acc_sh.at[dst_v.at[nwin - 1]], add=True)

        plsc.subcore_barrier()

        @pl.when(sid < n // _RB)
        def _():
            sl = pl.ds(sid * _RB, _RB)
            pltpu.sync_copy(acc_sh.at[sl], out_hbm.at[cid, sl])

    return agg_kernel(p, src3, dst3)


def _dinv_block(deg_ref, i):
    dparts = deg_ref[:, pl.ds(i * _RB, _RB), 0:1]
    deg = dparts[0] + dparts[1] + 1.0
    return lax.rsqrt(deg)


def _tc_matmul_scale(x, w, deg_parts):
    n, din = x.shape
    dh = w.shape[1]

    def body(x_ref, w_ref, deg_ref, o_ref):
        i = pl.program_id(0)
        h = lax.dot_general(
            x_ref[...], w_ref[...], (((1,), (0,)), ((), ())),
            precision=lax.Precision.HIGHEST, preferred_element_type=jnp.float32)
        o_ref[...] = h * _dinv_block(deg_ref, i)

    return pl.pallas_call(
        body,
        grid=(n // _RB,),
        in_specs=[
            pl.BlockSpec((_RB, din), lambda i: (i, 0)),
            pl.BlockSpec((din, dh), lambda i: (0, 0)),
            pl.BlockSpec((_NC, n, _DW), lambda i: (0, 0, 0)),
        ],
        out_specs=pl.BlockSpec((_RB, dh), lambda i: (i, 0)),
        out_shape=jax.ShapeDtypeStruct((n, dh), jnp.float32),
    )(x, w, deg_parts)


def _tc_mid(s_parts, p1, deg_parts, b1, w2):
    n, d = p1.shape
    dh = w2.shape[1]

    def body(s_ref, p_ref, deg_ref, b_ref, w_ref, o_ref):
        i = pl.program_id(0)
        dinv = _dinv_block(deg_ref, i)
        sv = s_ref[...]
        s = sv[0] + sv[1] - p_ref[...]
        t = jnp.maximum(s * dinv + b_ref[...], 0.0)
        h2 = lax.dot_general(
            t, w_ref[...], (((1,), (0,)), ((), ())),
            precision=lax.Precision.HIGHEST, preferred_element_type=jnp.float32)
        o_ref[...] = h2 * dinv

    return pl.pallas_call(
        body,
        grid=(n // _RB,),
        in_specs=[
            pl.BlockSpec((_NC, _RB, d), lambda i: (0, i, 0)),
            pl.BlockSpec((_RB, d), lambda i: (i, 0)),
            pl.BlockSpec((_NC, n, _DW), lambda i: (0, 0, 0)),
            pl.BlockSpec((1, d), lambda i: (0, 0)),
            pl.BlockSpec((d, dh), lambda i: (0, 0)),
        ],
        out_specs=pl.BlockSpec((_RB, dh), lambda i: (i, 0)),
        out_shape=jax.ShapeDtypeStruct((n, dh), jnp.float32),
    )(s_parts, p1, deg_parts, b1, w2)


def _tc_final(s_parts, p2, deg_parts, b2):
    n, d = p2.shape

    def body(s_ref, p_ref, deg_ref, b_ref, o_ref):
        i = pl.program_id(0)
        dinv = _dinv_block(deg_ref, i)
        sv = s_ref[...]
        s = sv[0] + sv[1] - p_ref[...]
        o_ref[...] = s * dinv + b_ref[...]

    return pl.pallas_call(
        body,
        grid=(n // _RB,),
        in_specs=[
            pl.BlockSpec((_NC, _RB, d), lambda i: (0, i, 0)),
            pl.BlockSpec((_RB, d), lambda i: (i, 0)),
            pl.BlockSpec((_NC, n, _DW), lambda i: (0, 0, 0)),
            pl.BlockSpec((1, d), lambda i: (0, 0)),
        ],
        out_specs=pl.BlockSpec((_RB, d), lambda i: (i, 0)),
        out_shape=jax.ShapeDtypeStruct((n, d), jnp.float32),
    )(s_parts, p2, deg_parts, b2)


def kernel(x, edge_index, W1, b1, W2, b2):
    n = x.shape[0]
    e = edge_index.shape[1]
    nwin = e // (_NW * _W)

    src3 = edge_index[0].reshape(_NW, nwin, _W)
    dst3 = edge_index[1].reshape(_NW, nwin, _W)
    zeros_col = jnp.zeros((n, _DW), jnp.float32)
    ones_col = jnp.ones((_W, _DW), jnp.float32)

    deg_parts = _sc_degree(
        dst3, zeros_col, ones_col,
        compiler_params=pltpu.CompilerParams(use_tc_tiling_on_sc=False),
    )
    p1 = _tc_matmul_scale(x, W1, deg_parts)
    s1 = _sc_aggregate(p1, src3, dst3)
    p2 = _tc_mid(s1, p1, deg_parts, b1.reshape(1, -1), W2)
    s2 = _sc_aggregate(p2, src3, dst3)
    return _tc_final(s2, p2, deg_parts, b2.reshape(1, -1))


# speedup vs baseline: 33.0721x; 1.1492x over previous
import functools

import jax
import jax.numpy as jnp
from jax import lax
from jax.experimental import pallas as pl
from jax.experimental.pallas import tpu as pltpu
from jax.experimental.pallas import tpu_sc as plsc

_NC = 2
_NS = 16
_NW = _NC * _NS
_W = 80
_RB = 1000


def _mesh():
    return plsc.VectorSubcoreMesh(core_axis_name="c", subcore_axis_name="s")


_DW = 16


def _sc_degree(dst3, zeros_col, ones_col, compiler_params=None):
    n, dw = zeros_col.shape
    nwin = dst3.shape[1]

    @functools.partial(
        pl.kernel,
        out_type=jax.ShapeDtypeStruct((_NC, n, dw), jnp.float32),
        mesh=_mesh(),
        compiler_params=compiler_params,
        scratch_types=[
            pltpu.VMEM_SHARED((n, dw), jnp.float32),
            pltpu.VMEM((nwin, _W), jnp.int32),
            pltpu.VMEM((_W, dw), jnp.float32),
        ],
    )
    def deg_kernel(dst_hbm, zeros_hbm, ones_hbm, out_hbm, acc_sh, dst_v, ones_v):
        cid = lax.axis_index("c")
        sid = lax.axis_index("s")
        wid = cid * _NS + sid

        pltpu.sync_copy(ones_hbm, ones_v)

        @pl.when(sid < n // _RB)
        def _():
            sl = pl.ds(sid * _RB, _RB)
            pltpu.sync_copy(zeros_hbm.at[sl], acc_sh.at[sl])

        plsc.subcore_barrier()

        pltpu.sync_copy(dst_hbm.at[wid], dst_v)

        @pl.loop(0, nwin)
        def _(w):
            pltpu.sync_copy(ones_v, acc_sh.at[dst_v.at[w]], add=True)

        plsc.subcore_barrier()

        @pl.when(sid < n // _RB)
        def _():
            sl = pl.ds(sid * _RB, _RB)
            pltpu.sync_copy(acc_sh.at[sl], out_hbm.at[cid, sl])

    return deg_kernel(dst3, zeros_col, ones_col)


def _sc_aggregate(p, src3, dst3):
    n, d = p.shape
    nwin = src3.shape[1]

    @functools.partial(
        pl.kernel,
        out_type=jax.ShapeDtypeStruct((_NC, n, d), jnp.float32),
        mesh=_mesh(),
        compiler_params=pltpu.CompilerParams(use_tc_tiling_on_sc=False),
        scratch_types=[
            pltpu.VMEM_SHARED((n, d), jnp.float32),
            pltpu.VMEM((nwin, _W), jnp.int32),
            pltpu.VMEM((nwin, _W), jnp.int32),
            pltpu.VMEM((_W, d), jnp.float32),
            pltpu.VMEM((_W, d), jnp.float32),
            pltpu.VMEM((_W, d), jnp.float32),
            pltpu.SemaphoreType.DMA,
            pltpu.SemaphoreType.DMA,
            pltpu.SemaphoreType.DMA,
        ],
    )
    def agg_kernel(p_hbm, src_hbm, dst_hbm, out_hbm, acc_sh, src_v, dst_v,
                   rows0, rows1, rows2, sem0, sem1, sem2):
        cid = lax.axis_index("c")
        sid = lax.axis_index("s")
        wid = cid * _NS + sid

        @pl.when(sid < n // _RB)
        def _():
            sl = pl.ds(sid * _RB, _RB)
            pltpu.sync_copy(p_hbm.at[sl], acc_sh.at[sl])

        pltpu.sync_copy(src_hbm.at[wid], src_v)
        pltpu.sync_copy(dst_hbm.at[wid], dst_v)
        plsc.subcore_barrier()

        bufs = ((rows0, sem0), (rows1, sem1), (rows2, sem2))
        nring = len(bufs)
        nwin_pad = -(-nwin // nring) * nring

        for b, (rv, sm) in enumerate(bufs):
            pltpu.async_copy(p_hbm.at[src_v.at[b]], rv, sm)

        @pl.loop(0, nwin_pad, step=nring)
        def _(w):
            for b, (rv, sm) in enumerate(bufs):
                wb = w + b

                @pl.when(wb < nwin)
                def _(wb=wb, rv=rv, sm=sm):
                    pltpu.make_async_copy(p_hbm.at[src_v.at[wb]], rv, sm).wait()
                    pltpu.sync_copy(rv, acc_sh.at[dst_v.at[wb]], add=True)

                    @pl.when(wb + nring < nwin)
                    def _():
                        pltpu.async_copy(p_hbm.at[src_v.at[wb + nring]], rv, sm)

        plsc.subcore_barrier()

        @pl.when(sid < n // _RB)
        def _():
            sl = pl.ds(sid * _RB, _RB)
            pltpu.sync_copy(acc_sh.at[sl], out_hbm.at[cid, sl])

    return agg_kernel(p, src3, dst3)


def _dinv_block(deg_ref, i):
    dparts = deg_ref[:, pl.ds(i * _RB, _RB), 0:1]
    deg = dparts[0] + dparts[1] + 1.0
    return lax.rsqrt(deg)


def _tc_matmul_scale(x, w, deg_parts):
    n, din = x.shape
    dh = w.shape[1]

    def body(x_ref, w_ref, deg_ref, o_ref):
        i = pl.program_id(0)
        h = lax.dot_general(
            x_ref[...], w_ref[...], (((1,), (0,)), ((), ())),
            precision=lax.Precision.HIGHEST, preferred_element_type=jnp.float32)
        o_ref[...] = h * _dinv_block(deg_ref, i)

    return pl.pallas_call(
        body,
        grid=(n // _RB,),
        in_specs=[
            pl.BlockSpec((_RB, din), lambda i: (i, 0)),
            pl.BlockSpec((din, dh), lambda i: (0, 0)),
            pl.BlockSpec((_NC, n, _DW), lambda i: (0, 0, 0)),
        ],
        out_specs=pl.BlockSpec((_RB, dh), lambda i: (i, 0)),
        out_shape=jax.ShapeDtypeStruct((n, dh), jnp.float32),
    )(x, w, deg_parts)


def _tc_mid(s_parts, p1, deg_parts, b1, w2):
    n, d = p1.shape
    dh = w2.shape[1]

    def body(s_ref, p_ref, deg_ref, b_ref, w_ref, o_ref):
        i = pl.program_id(0)
        dinv = _dinv_block(deg_ref, i)
        sv = s_ref[...]
        s = sv[0] + sv[1] - p_ref[...]
        t = jnp.maximum(s * dinv + b_ref[...], 0.0)
        h2 = lax.dot_general(
            t, w_ref[...], (((1,), (0,)), ((), ())),
            precision=lax.Precision.HIGHEST, preferred_element_type=jnp.float32)
        o_ref[...] = h2 * dinv

    return pl.pallas_call(
        body,
        grid=(n // _RB,),
        in_specs=[
            pl.BlockSpec((_NC, _RB, d), lambda i: (0, i, 0)),
            pl.BlockSpec((_RB, d), lambda i: (i, 0)),
            pl.BlockSpec((_NC, n, _DW), lambda i: (0, 0, 0)),
            pl.BlockSpec((1, d), lambda i: (0, 0)),
            pl.BlockSpec((d, dh), lambda i: (0, 0)),
        ],
        out_specs=pl.BlockSpec((_RB, dh), lambda i: (i, 0)),
        out_shape=jax.ShapeDtypeStruct((n, dh), jnp.float32),
    )(s_parts, p1, deg_parts, b1, w2)


def _tc_final(s_parts, p2, deg_parts, b2):
    n, d = p2.shape

    def body(s_ref, p_ref, deg_ref, b_ref, o_ref):
        i = pl.program_id(0)
        dinv = _dinv_block(deg_ref, i)
        sv = s_ref[...]
        s = sv[0] + sv[1] - p_ref[...]
        o_ref[...] = s * dinv + b_ref[...]

    return pl.pallas_call(
        body,
        grid=(n // _RB,),
        in_specs=[
            pl.BlockSpec((_NC, _RB, d), lambda i: (0, i, 0)),
            pl.BlockSpec((_RB, d), lambda i: (i, 0)),
            pl.BlockSpec((_NC, n, _DW), lambda i: (0, 0, 0)),
            pl.BlockSpec((1, d), lambda i: (0, 0)),
        ],
        out_specs=pl.BlockSpec((_RB, d), lambda i: (i, 0)),
        out_shape=jax.ShapeDtypeStruct((n, d), jnp.float32),
    )(s_parts, p2, deg_parts, b2)


def kernel(x, edge_index, W1, b1, W2, b2):
    n = x.shape[0]
    e = edge_index.shape[1]
    nwin = e // (_NW * _W)

    src3 = edge_index[0].reshape(_NW, nwin, _W)
    dst3 = edge_index[1].reshape(_NW, nwin, _W)
    zeros_col = jnp.zeros((n, _DW), jnp.float32)
    ones_col = jnp.ones((_W, _DW), jnp.float32)

    deg_parts = _sc_degree(
        dst3, zeros_col, ones_col,
        compiler_params=pltpu.CompilerParams(use_tc_tiling_on_sc=False),
    )
    p1 = _tc_matmul_scale(x, W1, deg_parts)
    s1 = _sc_aggregate(p1, src3, dst3)
    p2 = _tc_mid(s1, p1, deg_parts, b1.reshape(1, -1), W2)
    s2 = _sc_aggregate(p2, src3, dst3)
    return _tc_final(s2, p2, deg_parts, b2.reshape(1, -1))


# speedup vs baseline: 33.3285x; 1.0078x over previous
import functools

import jax
import jax.numpy as jnp
from jax import lax
from jax.experimental import pallas as pl
from jax.experimental.pallas import tpu as pltpu
from jax.experimental.pallas import tpu_sc as plsc

_NC = 2
_NS = 16
_NW = _NC * _NS
_W = 80
_RB = 1000


def _mesh():
    return plsc.VectorSubcoreMesh(core_axis_name="c", subcore_axis_name="s")


_DW = 16


def _sc_degree(dst3, zeros_col, ones_col, compiler_params=None):
    n, dw = zeros_col.shape
    nwin = dst3.shape[1]

    @functools.partial(
        pl.kernel,
        out_type=jax.ShapeDtypeStruct((_NC, n, dw), jnp.float32),
        mesh=_mesh(),
        compiler_params=compiler_params,
        scratch_types=[
            pltpu.VMEM_SHARED((n, dw), jnp.float32),
            pltpu.VMEM((nwin, _W), jnp.int32),
            pltpu.VMEM((_W, dw), jnp.float32),
        ],
    )
    def deg_kernel(dst_hbm, zeros_hbm, ones_hbm, out_hbm, acc_sh, dst_v, ones_v):
        cid = lax.axis_index("c")
        sid = lax.axis_index("s")
        wid = cid * _NS + sid

        pltpu.sync_copy(ones_hbm, ones_v)

        @pl.when(sid < n // _RB)
        def _():
            sl = pl.ds(sid * _RB, _RB)
            pltpu.sync_copy(zeros_hbm.at[sl], acc_sh.at[sl])

        plsc.subcore_barrier()

        pltpu.sync_copy(dst_hbm.at[wid], dst_v)

        @pl.loop(0, nwin)
        def _(w):
            pltpu.sync_copy(ones_v, acc_sh.at[dst_v.at[w]], add=True)

        plsc.subcore_barrier()

        @pl.when(sid < n // _RB)
        def _():
            sl = pl.ds(sid * _RB, _RB)
            pltpu.sync_copy(acc_sh.at[sl], out_hbm.at[cid, sl])

    return deg_kernel(dst3, zeros_col, ones_col)


def _sc_aggregate(p, src3, dst3):
    n, d = p.shape
    nwin = src3.shape[1]

    @functools.partial(
        pl.kernel,
        out_type=jax.ShapeDtypeStruct((_NC, n, d), jnp.float32),
        mesh=_mesh(),
        compiler_params=pltpu.CompilerParams(
            use_tc_tiling_on_sc=False,
            disable_bounds_checks=True,
            disable_semaphore_checks=True,
        ),
        scratch_types=[
            pltpu.VMEM_SHARED((n, d), jnp.float32),
            pltpu.VMEM((nwin, _W), jnp.int32),
            pltpu.VMEM((nwin, _W), jnp.int32),
            pltpu.VMEM((_W, d), jnp.float32),
            pltpu.VMEM((_W, d), jnp.float32),
            pltpu.VMEM((_W, d), jnp.float32),
            pltpu.SemaphoreType.DMA,
            pltpu.SemaphoreType.DMA,
            pltpu.SemaphoreType.DMA,
        ],
    )
    def agg_kernel(p_hbm, src_hbm, dst_hbm, out_hbm, acc_sh, src_v, dst_v,
                   rows0, rows1, rows2, sem0, sem1, sem2):
        cid = lax.axis_index("c")
        sid = lax.axis_index("s")
        wid = cid * _NS + sid

        @pl.when(sid < n // _RB)
        def _():
            sl = pl.ds(sid * _RB, _RB)
            pltpu.sync_copy(p_hbm.at[sl], acc_sh.at[sl])

        pltpu.sync_copy(src_hbm.at[wid], src_v)
        pltpu.sync_copy(dst_hbm.at[wid], dst_v)
        plsc.subcore_barrier()

        bufs = ((rows0, sem0), (rows1, sem1), (rows2, sem2))
        nring = len(bufs)
        nwin_pad = -(-nwin // nring) * nring

        for b, (rv, sm) in enumerate(bufs):
            pltpu.async_copy(p_hbm.at[src_v.at[b]], rv, sm)

        @pl.loop(0, nwin_pad, step=nring)
        def _(w):
            for b, (rv, sm) in enumerate(bufs):
                wb = w + b

                @pl.when(wb < nwin)
                def _(wb=wb, rv=rv, sm=sm):
                    pltpu.make_async_copy(p_hbm.at[src_v.at[wb]], rv, sm).wait()
                    pltpu.sync_copy(rv, acc_sh.at[dst_v.at[wb]], add=True)

                    @pl.when(wb + nring < nwin)
                    def _():
                        pltpu.async_copy(p_hbm.at[src_v.at[wb + nring]], rv, sm)

        plsc.subcore_barrier()

        @pl.when(sid < n // _RB)
        def _():
            sl = pl.ds(sid * _RB, _RB)
            pltpu.sync_copy(acc_sh.at[sl], out_hbm.at[cid, sl])

    return agg_kernel(p, src3, dst3)


def _dinv_block(deg_ref, i):
    dparts = deg_ref[:, pl.ds(i * _RB, _RB), 0:1]
    deg = dparts[0] + dparts[1] + 1.0
    return lax.rsqrt(deg)


def _tc_matmul(x, w):
    n, din = x.shape
    dh = w.shape[1]

    def body(x_ref, w_ref, o_ref):
        o_ref[...] = lax.dot_general(
            x_ref[...], w_ref[...], (((1,), (0,)), ((), ())),
            precision=lax.Precision.HIGHEST, preferred_element_type=jnp.float32)

    return pl.pallas_call(
        body,
        grid=(n // _RB,),
        in_specs=[
            pl.BlockSpec((_RB, din), lambda i: (i, 0)),
            pl.BlockSpec((din, dh), lambda i: (0, 0)),
        ],
        out_specs=pl.BlockSpec((_RB, dh), lambda i: (i, 0)),
        out_shape=jax.ShapeDtypeStruct((n, dh), jnp.float32),
    )(x, w)


def _tc_scale(h, deg_parts):
    n, d = h.shape

    def body(h_ref, deg_ref, o_ref):
        i = pl.program_id(0)
        o_ref[...] = h_ref[...] * _dinv_block(deg_ref, i)

    return pl.pallas_call(
        body,
        grid=(n // _RB,),
        in_specs=[
            pl.BlockSpec((_RB, d), lambda i: (i, 0)),
            pl.BlockSpec((_NC, n, _DW), lambda i: (0, 0, 0)),
        ],
        out_specs=pl.BlockSpec((_RB, d), lambda i: (i, 0)),
        out_shape=jax.ShapeDtypeStruct((n, d), jnp.float32),
    )(h, deg_parts)


def _tc_mid(s_parts, p1, deg_parts, b1, w2):
    n, d = p1.shape
    dh = w2.shape[1]

    def body(s_ref, p_ref, deg_ref, b_ref, w_ref, o_ref):
        i = pl.program_id(0)
        dinv = _dinv_block(deg_ref, i)
        sv = s_ref[...]
        s = sv[0] + sv[1] - p_ref[...]
        t = jnp.maximum(s * dinv + b_ref[...], 0.0)
        h2 = lax.dot_general(
            t, w_ref[...], (((1,), (0,)), ((), ())),
            precision=lax.Precision.HIGHEST, preferred_element_type=jnp.float32)
        o_ref[...] = h2 * dinv

    return pl.pallas_call(
        body,
        grid=(n // _RB,),
        in_specs=[
            pl.BlockSpec((_NC, _RB, d), lambda i: (0, i, 0)),
            pl.BlockSpec((_RB, d), lambda i: (i, 0)),
            pl.BlockSpec((_NC, n, _DW), lambda i: (0, 0, 0)),
            pl.BlockSpec((1, d), lambda i: (0, 0)),
            pl.BlockSpec((d, dh), lambda i: (0, 0)),
        ],
        out_specs=pl.BlockSpec((_RB, dh), lambda i: (i, 0)),
        out_shape=jax.ShapeDtypeStruct((n, dh), jnp.float32),
    )(s_parts, p1, deg_parts, b1, w2)


def _tc_final(s_parts, p2, deg_parts, b2):
    n, d = p2.shape

    def body(s_ref, p_ref, deg_ref, b_ref, o_ref):
        i = pl.program_id(0)
        dinv = _dinv_block(deg_ref, i)
        sv = s_ref[...]
        s = sv[0] + sv[1] - p_ref[...]
        o_ref[...] = s * dinv + b_ref[...]

    return pl.pallas_call(
        body,
        grid=(n // _RB,),
        in_specs=[
            pl.BlockSpec((_NC, _RB, d), lambda i: (0, i, 0)),
            pl.BlockSpec((_RB, d), lambda i: (i, 0)),
            pl.BlockSpec((_NC, n, _DW), lambda i: (0, 0, 0)),
            pl.BlockSpec((1, d), lambda i: (0, 0)),
        ],
        out_specs=pl.BlockSpec((_RB, d), lambda i: (i, 0)),
        out_shape=jax.ShapeDtypeStruct((n, d), jnp.float32),
    )(s_parts, p2, deg_parts, b2)


def kernel(x, edge_index, W1, b1, W2, b2):
    n = x.shape[0]
    e = edge_index.shape[1]
    nwin = e // (_NW * _W)

    src3 = edge_index[0].reshape(_NW, nwin, _W)
    dst3 = edge_index[1].reshape(_NW, nwin, _W)
    zeros_col = jnp.zeros((n, _DW), jnp.float32)
    ones_col = jnp.ones((_W, _DW), jnp.float32)

    deg_parts = _sc_degree(
        dst3, zeros_col, ones_col,
        compiler_params=pltpu.CompilerParams(
            use_tc_tiling_on_sc=False,
            disable_bounds_checks=True,
            disable_semaphore_checks=True,
        ),
    )
    h1 = _tc_matmul(x, W1)
    p1 = _tc_scale(h1, deg_parts)
    s1 = _sc_aggregate(p1, src3, dst3)
    p2 = _tc_mid(s1, p1, deg_parts, b1.reshape(1, -1), W2)
    s2 = _sc_aggregate(p2, src3, dst3)
    return _tc_final(s2, p2, deg_parts, b2.reshape(1, -1))
